# baseline (speedup 1.0000x reference)
import functools

import jax
import jax.numpy as jnp
from jax.experimental import pallas as pl
from jax.experimental.pallas import tpu as pltpu

_BN_EPS = 1e-5
_TN = 1024


def _nt_dot(a, b, prec=jax.lax.Precision.DEFAULT):
    return jax.lax.dot_general(
        a, b, (((1,), (1,)), ((), ())),
        preferred_element_type=jnp.float32,
        precision=prec)


def _knn_conv1_kernel(x1_ref, x2t_ref, sq1_ref, sq2_ref, f1t_ref, f2_ref,
                      w0_ref, b0_ref, z_ref, s_ref, ss_ref, *, d2):
    x1 = x1_ref[0]
    x2 = x2t_ref[0]
    sq1 = sq1_ref[0]
    sq2 = sq2_ref[0]
    inner = jax.lax.dot_general(
        x1, x2, (((1,), (0,)), ((), ())),
        preferred_element_type=jnp.float32,
        precision=jax.lax.Precision.DEFAULT)
    d = (sq1 + sq2) - 2.0 * inner
    inf = jnp.float32(jnp.inf)
    tn, p = d.shape
    iota = jax.lax.broadcasted_iota(jnp.int32, (tn, p), 1)
    dm = d
    vals, idxs = [], []
    for _ in range(3):
        mk = jnp.min(dm, axis=1, keepdims=True)
        jk = jnp.argmin(dm, axis=1, keepdims=True).astype(jnp.int32)
        vals.append(mk)
        idxs.append(jk)
        dm = jnp.where(iota == jk, inf, dm)
    rd0 = 1.0 / (vals[0] + 1e-8)
    rd1 = 1.0 / (vals[1] + 1e-8)
    rd2 = 1.0 / (vals[2] + 1e-8)
    norm = (rd0 + rd1) + rd2
    zero = jnp.float32(0.0)
    w = jnp.where(iota == idxs[0], rd0 / norm, zero)
    w += jnp.where(iota == idxs[1], rd1 / norm, zero)
    w += jnp.where(iota == idxs[2], rd2 / norm, zero)
    f2 = f2_ref[0]
    w_hi = w.astype(jnp.bfloat16).astype(jnp.float32)
    w_lo = w - w_hi
    f2_hi = f2.astype(jnp.bfloat16).astype(jnp.float32)
    f2_lo = f2 - f2_hi
    interp = _nt_dot(w_hi, f2_lo) + _nt_dot(w_lo, f2_hi)
    interp += _nt_dot(w_hi, f2_hi)
    zt = _nt_dot(interp, w0_ref[:, :d2])
    zt += jax.lax.dot_general(
        f1t_ref[0], w0_ref[:, d2:], (((0,), (1,)), ((), ())),
        preferred_element_type=jnp.float32)
    zt += b0_ref[...]
    z_ref[0] = zt

    first = (pl.program_id(0) == 0) & (pl.program_id(1) == 0)

    @pl.when(first)
    def _():
        s_ref[...] = jnp.zeros_like(s_ref)
        ss_ref[...] = jnp.zeros_like(ss_ref)

    s_ref[...] += jnp.sum(zt, axis=0, keepdims=True)
    ss_ref[...] += jnp.sum(zt * zt, axis=0, keepdims=True)


def _bn_conv2_kernel(z_ref, s_ref, ss_ref, g_ref, be_ref, w1_ref, b1_ref,
                     z1_ref, s1_ref, ss1_ref, *, inv_count):
    m = s_ref[...] * inv_count
    v = ss_ref[...] * inv_count - m * m
    scale = g_ref[...] / jnp.sqrt(v + _BN_EPS)
    shift = be_ref[...] - m * scale
    h = jnp.maximum(z_ref[0] * scale + shift, 0.0)
    z1 = _nt_dot(h, w1_ref[...]) + b1_ref[...]
    z1_ref[0] = z1

    first = (pl.program_id(0) == 0) & (pl.program_id(1) == 0)

    @pl.when(first)
    def _():
        s1_ref[...] = jnp.zeros_like(s1_ref)
        ss1_ref[...] = jnp.zeros_like(ss1_ref)

    s1_ref[...] += jnp.sum(z1, axis=0, keepdims=True)
    ss1_ref[...] += jnp.sum(z1 * z1, axis=0, keepdims=True)


def _bn_out_kernel(z1_ref, s_ref, ss_ref, g_ref, be_ref, o_ref, *, inv_count):
    m = s_ref[...] * inv_count
    v = ss_ref[...] * inv_count - m * m
    scale = g_ref[...] / jnp.sqrt(v + _BN_EPS)
    shift = be_ref[...] - m * scale
    o_ref[0] = jnp.transpose(jnp.maximum(z1_ref[0] * scale + shift, 0.0),
                             (1, 0))


def kernel(xyz1, xyz2, feat1, feat2, W0, b0, g0, be0, W1, b1, g1, be1):
    B, N, C = xyz1.shape
    P = xyz2.shape[1]
    D1 = feat1.shape[1]
    D2 = feat2.shape[1]
    H0 = W0.shape[0]
    H1 = W1.shape[0]
    TN = _TN
    NT = N // TN
    inv_count = 1.0 / (B * N)

    xyz2t = jnp.transpose(xyz2, (0, 2, 1))
    sq1 = jnp.sum(xyz1 * xyz1, axis=-1).reshape(B, N, 1)
    sq2 = jnp.sum(xyz2 * xyz2, axis=-1).reshape(B, 1, P)
    row = lambda v: v.reshape(1, -1)

    seq = pltpu.CompilerParams(dimension_semantics=("arbitrary", "arbitrary"))
    grid = (B, NT)

    z0t, s0, ss0 = pl.pallas_call(
        functools.partial(_knn_conv1_kernel, d2=D2),
        grid=grid,
        in_specs=[
            pl.BlockSpec((1, TN, C), lambda b, n: (b, n, 0)),
            pl.BlockSpec((1, C, P), lambda b, n: (b, 0, 0)),
            pl.BlockSpec((1, TN, 1), lambda b, n: (b, n, 0)),
            pl.BlockSpec((1, 1, P), lambda b, n: (b, 0, 0)),
            pl.BlockSpec((1, D1, TN), lambda b, n: (b, 0, n)),
            pl.BlockSpec((1, D2, P), lambda b, n: (b, 0, 0)),
            pl.BlockSpec((H0, D2 + D1), lambda b, n: (0, 0)),
            pl.BlockSpec((1, H0), lambda b, n: (0, 0)),
        ],
        out_specs=[
            pl.BlockSpec((1, TN, H0), lambda b, n: (b, n, 0)),
            pl.BlockSpec((1, H0), lambda b, n: (0, 0)),
            pl.BlockSpec((1, H0), lambda b, n: (0, 0)),
        ],
        out_shape=[
            jax.ShapeDtypeStruct((B, N, H0), jnp.float32),
            jax.ShapeDtypeStruct((1, H0), jnp.float32),
            jax.ShapeDtypeStruct((1, H0), jnp.float32),
        ],
        compiler_params=seq,
    )(xyz1, xyz2t, sq1, sq2, feat1, feat2, W0, row(b0))

    z1t, s1, ss1 = pl.pallas_call(
        functools.partial(_bn_conv2_kernel, inv_count=inv_count),
        grid=grid,
        in_specs=[
            pl.BlockSpec((1, TN, H0), lambda b, n: (b, n, 0)),
            pl.BlockSpec((1, H0), lambda b, n: (0, 0)),
            pl.BlockSpec((1, H0), lambda b, n: (0, 0)),
            pl.BlockSpec((1, H0), lambda b, n: (0, 0)),
            pl.BlockSpec((1, H0), lambda b, n: (0, 0)),
            pl.BlockSpec((H1, H0), lambda b, n: (0, 0)),
            pl.BlockSpec((1, H1), lambda b, n: (0, 0)),
        ],
        out_specs=[
            pl.BlockSpec((1, TN, H1), lambda b, n: (b, n, 0)),
            pl.BlockSpec((1, H1), lambda b, n: (0, 0)),
            pl.BlockSpec((1, H1), lambda b, n: (0, 0)),
        ],
        out_shape=[
            jax.ShapeDtypeStruct((B, N, H1), jnp.float32),
            jax.ShapeDtypeStruct((1, H1), jnp.float32),
            jax.ShapeDtypeStruct((1, H1), jnp.float32),
        ],
        compiler_params=seq,
    )(z0t, s0, ss0, row(g0), row(be0), W1, row(b1))

    outt = pl.pallas_call(
        functools.partial(_bn_out_kernel, inv_count=inv_count),
        grid=grid,
        in_specs=[
            pl.BlockSpec((1, TN, H1), lambda b, n: (b, n, 0)),
            pl.BlockSpec((1, H1), lambda b, n: (0, 0)),
            pl.BlockSpec((1, H1), lambda b, n: (0, 0)),
            pl.BlockSpec((1, H1), lambda b, n: (0, 0)),
            pl.BlockSpec((1, H1), lambda b, n: (0, 0)),
        ],
        out_specs=pl.BlockSpec((1, H1, TN), lambda b, n: (b, 0, n)),
        out_shape=jax.ShapeDtypeStruct((B, H1, N), jnp.float32),
        compiler_params=seq,
    )(z1t, s1, ss1, row(g1), row(be1))

    return outt


# speedup vs baseline: 1.3154x; 1.3154x over previous
import functools

import jax
import jax.numpy as jnp
from jax.experimental import pallas as pl
from jax.experimental.pallas import tpu as pltpu

_BN_EPS = 1e-5
_TN = 2048


def _nt_dot(a, b, prec=jax.lax.Precision.DEFAULT):
    return jax.lax.dot_general(
        a, b, (((1,), (1,)), ((), ())),
        preferred_element_type=jnp.float32,
        precision=prec)


def _knn_conv1_kernel(x1_ref, x2t_ref, sq1_ref, sq2_ref, f1t_ref, f2_ref,
                      w0_ref, b0_ref, z_ref, s_ref, ss_ref, *, d2):
    x1 = x1_ref[0]
    x2 = x2t_ref[0]
    sq1 = sq1_ref[0]
    sq2 = sq2_ref[0]
    inner = jax.lax.dot_general(
        x1, x2, (((1,), (0,)), ((), ())),
        preferred_element_type=jnp.float32,
        precision=jax.lax.Precision.DEFAULT)
    d = (sq1 + sq2) - 2.0 * inner
    inf = jnp.float32(jnp.inf)
    tn, p = d.shape
    iota = jax.lax.broadcasted_iota(jnp.int32, (tn, p), 1)
    dm = d
    vals, idxs = [], []
    for _ in range(3):
        mk = jnp.min(dm, axis=1, keepdims=True)
        jk = jnp.min(jnp.where(dm == mk, iota, p), axis=1, keepdims=True)
        vals.append(mk)
        idxs.append(jk)
        dm = jnp.where(iota == jk, inf, dm)
    rd0 = 1.0 / (vals[0] + 1e-8)
    rd1 = 1.0 / (vals[1] + 1e-8)
    rd2 = 1.0 / (vals[2] + 1e-8)
    norm = (rd0 + rd1) + rd2
    zero = jnp.float32(0.0)
    w = jnp.where(iota == idxs[0], rd0 / norm, zero)
    w += jnp.where(iota == idxs[1], rd1 / norm, zero)
    w += jnp.where(iota == idxs[2], rd2 / norm, zero)
    f2 = f2_ref[0]
    w_hi = w.astype(jnp.bfloat16).astype(jnp.float32)
    w_lo = w - w_hi
    f2_hi = f2.astype(jnp.bfloat16).astype(jnp.float32)
    f2_lo = f2 - f2_hi
    interp = _nt_dot(w_hi, f2_lo) + _nt_dot(w_lo, f2_hi)
    interp += _nt_dot(w_hi, f2_hi)
    zt = _nt_dot(interp, w0_ref[:, :d2])
    zt += jax.lax.dot_general(
        f1t_ref[0], w0_ref[:, d2:], (((0,), (1,)), ((), ())),
        preferred_element_type=jnp.float32)
    zt += b0_ref[...]
    z_ref[0] = zt

    first = (pl.program_id(0) == 0) & (pl.program_id(1) == 0)

    @pl.when(first)
    def _():
        s_ref[...] = jnp.zeros_like(s_ref)
        ss_ref[...] = jnp.zeros_like(ss_ref)

    s_ref[...] += jnp.sum(zt, axis=0, keepdims=True)
    ss_ref[...] += jnp.sum(zt * zt, axis=0, keepdims=True)


def _bn_conv2_kernel(z_ref, s_ref, ss_ref, g_ref, be_ref, w1_ref, b1_ref,
                     z1_ref, s1_ref, ss1_ref, *, inv_count):
    m = s_ref[...] * inv_count
    v = ss_ref[...] * inv_count - m * m
    scale = g_ref[...] / jnp.sqrt(v + _BN_EPS)
    shift = be_ref[...] - m * scale
    h = jnp.maximum(z_ref[0] * scale + shift, 0.0)
    z1 = _nt_dot(h, w1_ref[...]) + b1_ref[...]
    z1_ref[0] = z1

    first = (pl.program_id(0) == 0) & (pl.program_id(1) == 0)

    @pl.when(first)
    def _():
        s1_ref[...] = jnp.zeros_like(s1_ref)
        ss1_ref[...] = jnp.zeros_like(ss1_ref)

    s1_ref[...] += jnp.sum(z1, axis=0, keepdims=True)
    ss1_ref[...] += jnp.sum(z1 * z1, axis=0, keepdims=True)


def _bn_out_kernel(z1_ref, s_ref, ss_ref, g_ref, be_ref, o_ref, *, inv_count):
    m = s_ref[...] * inv_count
    v = ss_ref[...] * inv_count - m * m
    scale = g_ref[...] / jnp.sqrt(v + _BN_EPS)
    shift = be_ref[...] - m * scale
    o_ref[0] = jnp.transpose(jnp.maximum(z1_ref[0] * scale + shift, 0.0),
                             (1, 0))


def kernel(xyz1, xyz2, feat1, feat2, W0, b0, g0, be0, W1, b1, g1, be1):
    B, N, C = xyz1.shape
    P = xyz2.shape[1]
    D1 = feat1.shape[1]
    D2 = feat2.shape[1]
    H0 = W0.shape[0]
    H1 = W1.shape[0]
    TN = _TN
    NT = N // TN
    inv_count = 1.0 / (B * N)

    xyz2t = jnp.transpose(xyz2, (0, 2, 1))
    sq1 = jnp.sum(xyz1 * xyz1, axis=-1).reshape(B, N, 1)
    sq2 = jnp.sum(xyz2 * xyz2, axis=-1).reshape(B, 1, P)
    row = lambda v: v.reshape(1, -1)

    seq = pltpu.CompilerParams(dimension_semantics=("arbitrary", "arbitrary"))
    grid = (B, NT)

    z0t, s0, ss0 = pl.pallas_call(
        functools.partial(_knn_conv1_kernel, d2=D2),
        grid=grid,
        in_specs=[
            pl.BlockSpec((1, TN, C), lambda b, n: (b, n, 0)),
            pl.BlockSpec((1, C, P), lambda b, n: (b, 0, 0)),
            pl.BlockSpec((1, TN, 1), lambda b, n: (b, n, 0)),
            pl.BlockSpec((1, 1, P), lambda b, n: (b, 0, 0)),
            pl.BlockSpec((1, D1, TN), lambda b, n: (b, 0, n)),
            pl.BlockSpec((1, D2, P), lambda b, n: (b, 0, 0)),
            pl.BlockSpec((H0, D2 + D1), lambda b, n: (0, 0)),
            pl.BlockSpec((1, H0), lambda b, n: (0, 0)),
        ],
        out_specs=[
            pl.BlockSpec((1, TN, H0), lambda b, n: (b, n, 0)),
            pl.BlockSpec((1, H0), lambda b, n: (0, 0)),
            pl.BlockSpec((1, H0), lambda b, n: (0, 0)),
        ],
        out_shape=[
            jax.ShapeDtypeStruct((B, N, H0), jnp.float32),
            jax.ShapeDtypeStruct((1, H0), jnp.float32),
            jax.ShapeDtypeStruct((1, H0), jnp.float32),
        ],
        compiler_params=seq,
    )(xyz1, xyz2t, sq1, sq2, feat1, feat2, W0, row(b0))

    z1t, s1, ss1 = pl.pallas_call(
        functools.partial(_bn_conv2_kernel, inv_count=inv_count),
        grid=grid,
        in_specs=[
            pl.BlockSpec((1, TN, H0), lambda b, n: (b, n, 0)),
            pl.BlockSpec((1, H0), lambda b, n: (0, 0)),
            pl.BlockSpec((1, H0), lambda b, n: (0, 0)),
            pl.BlockSpec((1, H0), lambda b, n: (0, 0)),
            pl.BlockSpec((1, H0), lambda b, n: (0, 0)),
            pl.BlockSpec((H1, H0), lambda b, n: (0, 0)),
            pl.BlockSpec((1, H1), lambda b, n: (0, 0)),
        ],
        out_specs=[
            pl.BlockSpec((1, TN, H1), lambda b, n: (b, n, 0)),
            pl.BlockSpec((1, H1), lambda b, n: (0, 0)),
            pl.BlockSpec((1, H1), lambda b, n: (0, 0)),
        ],
        out_shape=[
            jax.ShapeDtypeStruct((B, N, H1), jnp.float32),
            jax.ShapeDtypeStruct((1, H1), jnp.float32),
            jax.ShapeDtypeStruct((1, H1), jnp.float32),
        ],
        compiler_params=seq,
    )(z0t, s0, ss0, row(g0), row(be0), W1, row(b1))

    outt = pl.pallas_call(
        functools.partial(_bn_out_kernel, inv_count=inv_count),
        grid=grid,
        in_specs=[
            pl.BlockSpec((1, TN, H1), lambda b, n: (b, n, 0)),
            pl.BlockSpec((1, H1), lambda b, n: (0, 0)),
            pl.BlockSpec((1, H1), lambda b, n: (0, 0)),
            pl.BlockSpec((1, H1), lambda b, n: (0, 0)),
            pl.BlockSpec((1, H1), lambda b, n: (0, 0)),
        ],
        out_specs=pl.BlockSpec((1, H1, TN), lambda b, n: (b, 0, n)),
        out_shape=jax.ShapeDtypeStruct((B, H1, N), jnp.float32),
        compiler_params=seq,
    )(z1t, s1, ss1, row(g1), row(be1))

    return outt


# speedup vs baseline: 1.3496x; 1.0260x over previous
import functools

import jax
import jax.numpy as jnp
from jax.experimental import pallas as pl
from jax.experimental.pallas import tpu as pltpu

_BN_EPS = 1e-5
_TN = 2048


def _nt_dot(a, b, prec=jax.lax.Precision.DEFAULT):
    return jax.lax.dot_general(
        a, b, (((1,), (1,)), ((), ())),
        preferred_element_type=jnp.float32,
        precision=prec)


def _knn_conv1_kernel(x1_ref, x2t_ref, sq1_ref, sq2_ref, f1t_ref, f2_ref,
                      w0_ref, b0_ref, z_ref, s_ref, ss_ref, *, d2):
    x1 = x1_ref[0]
    x2 = x2t_ref[0]
    sq1 = sq1_ref[0]
    sq2 = sq2_ref[0]
    inner = jax.lax.dot_general(
        x1, x2, (((1,), (0,)), ((), ())),
        preferred_element_type=jnp.float32,
        precision=jax.lax.Precision.DEFAULT)
    d = (sq1 + sq2) - 2.0 * inner
    inf = jnp.float32(jnp.inf)
    tn, p = d.shape
    iota = jax.lax.broadcasted_iota(jnp.int32, (tn, p), 1)
    dm = d
    vals = []
    for _ in range(3):
        mk = jnp.min(dm, axis=1, keepdims=True)
        jk = jnp.min(jnp.where(dm == mk, iota, p), axis=1, keepdims=True)
        vals.append(mk)
        dm = jnp.where(iota == jk, inf, dm)
    rd0 = 1.0 / (vals[0] + 1e-8)
    rd1 = 1.0 / (vals[1] + 1e-8)
    rd2 = 1.0 / (vals[2] + 1e-8)
    norm = (rd0 + rd1) + rd2
    w = jnp.where(dm != d, (1.0 / (d + 1e-8)) / norm, jnp.float32(0.0))
    f2 = f2_ref[0]
    w_hi = w.astype(jnp.bfloat16).astype(jnp.float32)
    w_lo = w - w_hi
    f2_hi = f2.astype(jnp.bfloat16).astype(jnp.float32)
    f2_lo = f2 - f2_hi
    interp = _nt_dot(w_hi, f2_lo) + _nt_dot(w_lo, f2_hi)
    interp += _nt_dot(w_hi, f2_hi)
    zt = _nt_dot(interp, w0_ref[:, :d2])
    zt += jax.lax.dot_general(
        f1t_ref[0], w0_ref[:, d2:], (((0,), (1,)), ((), ())),
        preferred_element_type=jnp.float32)
    zt += b0_ref[...]
    z_ref[0] = zt

    first = (pl.program_id(0) == 0) & (pl.program_id(1) == 0)

    @pl.when(first)
    def _():
        s_ref[...] = jnp.zeros_like(s_ref)
        ss_ref[...] = jnp.zeros_like(ss_ref)

    s_ref[...] += jnp.sum(zt, axis=0, keepdims=True)
    ss_ref[...] += jnp.sum(zt * zt, axis=0, keepdims=True)


def _bn_conv2_kernel(z_ref, s_ref, ss_ref, g_ref, be_ref, w1_ref, b1_ref,
                     z1_ref, s1_ref, ss1_ref, *, inv_count):
    m = s_ref[...] * inv_count
    v = ss_ref[...] * inv_count - m * m
    scale = g_ref[...] / jnp.sqrt(v + _BN_EPS)
    shift = be_ref[...] - m * scale
    h = jnp.maximum(z_ref[0] * scale + shift, 0.0)
    z1 = _nt_dot(h, w1_ref[...]) + b1_ref[...]
    z1_ref[0] = z1

    first = (pl.program_id(0) == 0) & (pl.program_id(1) == 0)

    @pl.when(first)
    def _():
        s1_ref[...] = jnp.zeros_like(s1_ref)
        ss1_ref[...] = jnp.zeros_like(ss1_ref)

    s1_ref[...] += jnp.sum(z1, axis=0, keepdims=True)
    ss1_ref[...] += jnp.sum(z1 * z1, axis=0, keepdims=True)


def _bn_out_kernel(z1_ref, s_ref, ss_ref, g_ref, be_ref, o_ref, *, inv_count):
    m = s_ref[...] * inv_count
    v = ss_ref[...] * inv_count - m * m
    scale = g_ref[...] / jnp.sqrt(v + _BN_EPS)
    shift = be_ref[...] - m * scale
    o_ref[0] = jnp.transpose(jnp.maximum(z1_ref[0] * scale + shift, 0.0),
                             (1, 0))


def kernel(xyz1, xyz2, feat1, feat2, W0, b0, g0, be0, W1, b1, g1, be1):
    B, N, C = xyz1.shape
    P = xyz2.shape[1]
    D1 = feat1.shape[1]
    D2 = feat2.shape[1]
    H0 = W0.shape[0]
    H1 = W1.shape[0]
    TN = _TN
    NT = N // TN
    inv_count = 1.0 / (B * N)

    xyz2t = jnp.transpose(xyz2, (0, 2, 1))
    sq1 = jnp.sum(xyz1 * xyz1, axis=-1).reshape(B, N, 1)
    sq2 = jnp.sum(xyz2 * xyz2, axis=-1).reshape(B, 1, P)
    row = lambda v: v.reshape(1, -1)

    seq = pltpu.CompilerParams(dimension_semantics=("arbitrary", "arbitrary"))
    grid = (B, NT)

    z0t, s0, ss0 = pl.pallas_call(
        functools.partial(_knn_conv1_kernel, d2=D2),
        grid=grid,
        in_specs=[
            pl.BlockSpec((1, TN, C), lambda b, n: (b, n, 0)),
            pl.BlockSpec((1, C, P), lambda b, n: (b, 0, 0)),
            pl.BlockSpec((1, TN, 1), lambda b, n: (b, n, 0)),
            pl.BlockSpec((1, 1, P), lambda b, n: (b, 0, 0)),
            pl.BlockSpec((1, D1, TN), lambda b, n: (b, 0, n)),
            pl.BlockSpec((1, D2, P), lambda b, n: (b, 0, 0)),
            pl.BlockSpec((H0, D2 + D1), lambda b, n: (0, 0)),
            pl.BlockSpec((1, H0), lambda b, n: (0, 0)),
        ],
        out_specs=[
            pl.BlockSpec((1, TN, H0), lambda b, n: (b, n, 0)),
            pl.BlockSpec((1, H0), lambda b, n: (0, 0)),
            pl.BlockSpec((1, H0), lambda b, n: (0, 0)),
        ],
        out_shape=[
            jax.ShapeDtypeStruct((B, N, H0), jnp.float32),
            jax.ShapeDtypeStruct((1, H0), jnp.float32),
            jax.ShapeDtypeStruct((1, H0), jnp.float32),
        ],
        compiler_params=seq,
    )(xyz1, xyz2t, sq1, sq2, feat1, feat2, W0, row(b0))

    z1t, s1, ss1 = pl.pallas_call(
        functools.partial(_bn_conv2_kernel, inv_count=inv_count),
        grid=grid,
        in_specs=[
            pl.BlockSpec((1, TN, H0), lambda b, n: (b, n, 0)),
            pl.BlockSpec((1, H0), lambda b, n: (0, 0)),
            pl.BlockSpec((1, H0), lambda b, n: (0, 0)),
            pl.BlockSpec((1, H0), lambda b, n: (0, 0)),
            pl.BlockSpec((1, H0), lambda b, n: (0, 0)),
            pl.BlockSpec((H1, H0), lambda b, n: (0, 0)),
            pl.BlockSpec((1, H1), lambda b, n: (0, 0)),
        ],
        out_specs=[
            pl.BlockSpec((1, TN, H1), lambda b, n: (b, n, 0)),
            pl.BlockSpec((1, H1), lambda b, n: (0, 0)),
            pl.BlockSpec((1, H1), lambda b, n: (0, 0)),
        ],
        out_shape=[
            jax.ShapeDtypeStruct((B, N, H1), jnp.float32),
            jax.ShapeDtypeStruct((1, H1), jnp.float32),
            jax.ShapeDtypeStruct((1, H1), jnp.float32),
        ],
        compiler_params=seq,
    )(z0t, s0, ss0, row(g0), row(be0), W1, row(b1))

    outt = pl.pallas_call(
        functools.partial(_bn_out_kernel, inv_count=inv_count),
        grid=grid,
        in_specs=[
            pl.BlockSpec((1, TN, H1), lambda b, n: (b, n, 0)),
            pl.BlockSpec((1, H1), lambda b, n: (0, 0)),
            pl.BlockSpec((1, H1), lambda b, n: (0, 0)),
            pl.BlockSpec((1, H1), lambda b, n: (0, 0)),
            pl.BlockSpec((1, H1), lambda b, n: (0, 0)),
        ],
        out_specs=pl.BlockSpec((1, H1, TN), lambda b, n: (b, 0, n)),
        out_shape=jax.ShapeDtypeStruct((B, H1, N), jnp.float32),
        compiler_params=seq,
    )(z1t, s1, ss1, row(g1), row(be1))

    return outt
